# baseline (speedup 1.0000x reference)
import functools

import jax
import jax.numpy as jnp
from jax import lax
from jax.experimental import pallas as pl
from jax.experimental.pallas import tpu as pltpu
from jax.experimental.pallas import tpu_sc as plsc

N = 10000
D = 128
E = 320000
NC = 2
NS = 16
NW = NC * NS
K = 128
EPW = E // NW
NH = 3
CH = 26
TAIL = EPW - NH * CH * K
ROWS_PT = 640
NPAD = ROWS_PT * NS
LAST = N - (NS - 1) * ROWS_PT

_MESH = plsc.VectorSubcoreMesh(core_axis_name="c", subcore_axis_name="s")


def _sc_agg_body(with_deg, h_hbm, src_hbm, dst_hbm, src16_hbm, dst16_hbm,
                 zer_hbm, agg_hbm, *rest):
    if with_deg:
        (deg_hbm, src_v, dst_v, src16_v, dst16_v, rows_v, rows16_v,
         ones_v, zdeg_v, agg_s, deg_s, g0, g1, s0, s1) = rest
    else:
        (src_v, dst_v, src16_v, dst16_v, rows_v, rows16_v,
         agg_s, g0, g1, s0, s1) = rest

    c = lax.axis_index("c")
    s = lax.axis_index("s")
    wid = c * NS + s
    base = s * ROWS_PT

    pltpu.sync_copy(zer_hbm, agg_s.at[pl.ds(base, ROWS_PT)])
    if with_deg:
        def _fill_ones(i, _):
            ones_v[pl.ds(i * 16, 16)] = jnp.full((16,), 1.0, jnp.float32)
            return 0
        lax.fori_loop(0, K // 16, _fill_ones, 0)

        def _fill_zdeg(i, _):
            zdeg_v[pl.ds(i * 16, 16)] = jnp.zeros((16,), jnp.float32)
            return 0
        lax.fori_loop(0, ROWS_PT // 16, _fill_zdeg, 0)
        pltpu.sync_copy(zdeg_v, deg_s.at[pl.ds(base, ROWS_PT)])
    plsc.subcore_barrier()

    gsems = (g0, g1)
    ssems = (s0, s1)
    for seg in range(NH):
        pltpu.sync_copy(src_hbm.at[wid, seg], src_v)
        pltpu.sync_copy(dst_hbm.at[wid, seg], dst_v)
        pltpu.async_copy(h_hbm.at[src_v.at[0]], rows_v.at[0], g0)
        pltpu.async_copy(h_hbm.at[src_v.at[1]], rows_v.at[1], g1)

        def _cycle(g, _):
            for b in range(2):
                j = g * 2 + b
                pltpu.make_async_copy(h_hbm.at[pl.ds(0, K)], rows_v.at[b],
                                      gsems[b]).wait()
                sd = pltpu.async_copy(rows_v.at[b], agg_s.at[dst_v.at[j]],
                                      ssems[b], add=True)
                if with_deg:
                    pltpu.sync_copy(ones_v, deg_s.at[dst_v.at[j]], add=True)
                sd.wait()

                @pl.when(j + 2 < CH)
                def _():
                    pltpu.async_copy(h_hbm.at[src_v.at[j + 2]], rows_v.at[b],
                                     gsems[b])
            return 0
        lax.fori_loop(0, CH // 2, _cycle, 0)

    pltpu.sync_copy(src16_hbm.at[wid], src16_v)
    pltpu.sync_copy(dst16_hbm.at[wid], dst16_v)
    pltpu.async_copy(h_hbm.at[src16_v], rows16_v, g1).wait()
    pltpu.sync_copy(rows16_v, agg_s.at[dst16_v], add=True)
    if with_deg:
        pltpu.sync_copy(ones_v.at[pl.ds(0, TAIL)], deg_s.at[dst16_v], add=True)
    plsc.subcore_barrier()

    if with_deg:
        pltpu.sync_copy(deg_s.at[pl.ds(base, ROWS_PT)], zdeg_v)

    @pl.when(s < NS - 1)
    def _():
        pltpu.sync_copy(agg_s.at[pl.ds(base, ROWS_PT)],
                        agg_hbm.at[c, pl.ds(base, ROWS_PT)])
        if with_deg:
            pltpu.sync_copy(zdeg_v, deg_hbm.at[pl.ds(c * N + base, ROWS_PT)])

    @pl.when(s == NS - 1)
    def _():
        pltpu.sync_copy(agg_s.at[pl.ds(base, LAST)],
                        agg_hbm.at[c, pl.ds(base, LAST)])
        if with_deg:
            pltpu.sync_copy(zdeg_v.at[pl.ds(0, LAST)],
                            deg_hbm.at[pl.ds(c * N + base, LAST)])


_IDX_SCRATCH = [
    pltpu.VMEM((CH, K), jnp.int32),
    pltpu.VMEM((CH, K), jnp.int32),
    pltpu.VMEM((TAIL,), jnp.int32),
    pltpu.VMEM((TAIL,), jnp.int32),
    pltpu.VMEM((2, K, D), jnp.float32),
    pltpu.VMEM((TAIL, D), jnp.float32),
]

_sc_agg_deg = functools.partial(
    pl.kernel,
    out_type=(jax.ShapeDtypeStruct((NC, N, D), jnp.float32),
              jax.ShapeDtypeStruct((NC * N,), jnp.float32)),
    mesh=_MESH,
    scratch_types=_IDX_SCRATCH + [
        pltpu.VMEM((K,), jnp.float32),
        pltpu.VMEM((ROWS_PT,), jnp.float32),
        pltpu.VMEM_SHARED((NPAD, D), jnp.float32),
        pltpu.VMEM_SHARED((NPAD,), jnp.float32),
    ] + [pltpu.SemaphoreType.DMA] * 4,
)(functools.partial(_sc_agg_body, True))

_sc_agg_nodeg = functools.partial(
    pl.kernel,
    out_type=jax.ShapeDtypeStruct((NC, N, D), jnp.float32),
    mesh=_MESH,
    scratch_types=_IDX_SCRATCH + [
        pltpu.VMEM_SHARED((NPAD, D), jnp.float32),
    ] + [pltpu.SemaphoreType.DMA] * 4,
)(functools.partial(_sc_agg_body, False))


R = 1000


def _tc_layer_body(relu, agg_ref, degt_ref, h_ref, wl_ref, bl_ref, wr_ref, o_ref):
    aggsum = agg_ref[0] + agg_ref[1]
    deg = degt_ref[:, 0] + degt_ref[:, 1]
    invd = 1.0 / jnp.maximum(deg, 1.0)
    m = aggsum * invd[:, None]
    out = lax.dot_general(m, wl_ref[...], (((1,), (1,)), ((), ())),
                          preferred_element_type=jnp.float32)
    out = out + bl_ref[...]
    out = out + lax.dot_general(h_ref[...], wr_ref[...], (((1,), (1,)), ((), ())),
                                preferred_element_type=jnp.float32)
    if relu:
        out = jnp.maximum(out, 0.0)
    o_ref[...] = out


def _tc_layer(relu):
    return pl.pallas_call(
        functools.partial(_tc_layer_body, relu),
        grid=(N // R,),
        in_specs=[
            pl.BlockSpec((NC, R, D), lambda i: (0, i, 0)),
            pl.BlockSpec((R, NC), lambda i: (i, 0)),
            pl.BlockSpec((R, D), lambda i: (i, 0)),
            pl.BlockSpec((D, D), lambda i: (0, 0)),
            pl.BlockSpec((1, D), lambda i: (0, 0)),
            pl.BlockSpec((D, D), lambda i: (0, 0)),
        ],
        out_specs=pl.BlockSpec((R, D), lambda i: (i, 0)),
        out_shape=jax.ShapeDtypeStruct((N, D), jnp.float32),
    )


def kernel(x, edge_index, Wl1, bl1, Wr1, Wl2, bl2, Wr2, Wl3, bl3, Wr3,
           Wl4, bl4, Wr4, Wl5, bl5, Wr5):
    src2 = edge_index[0].reshape(NW, EPW)
    dst2 = edge_index[1].reshape(NW, EPW)
    full = NH * CH * K
    srcp = src2[:, :full].reshape(NW, NH, CH, K)
    dstp = dst2[:, :full].reshape(NW, NH, CH, K)
    src16 = src2[:, full:]
    dst16 = dst2[:, full:]
    zer = jnp.zeros((ROWS_PT, D), jnp.float32)

    layers = [(Wl1, bl1, Wr1), (Wl2, bl2, Wr2), (Wl3, bl3, Wr3),
              (Wl4, bl4, Wr4), (Wl5, bl5, Wr5)]
    h = x
    degt = None
    for i, (Wl, bl, Wr) in enumerate(layers):
        if i == 0:
            agg, deg = _sc_agg_deg(h, srcp, dstp, src16, dst16, zer)
            degt = deg.reshape(NC, N).T
        else:
            agg = _sc_agg_nodeg(h, srcp, dstp, src16, dst16, zer)
        h = _tc_layer(i < 4)(agg, degt, h, Wl, bl.reshape(1, D), Wr)
    return h


# speedup vs baseline: 1.0409x; 1.0409x over previous
import functools

import jax
import jax.numpy as jnp
from jax import lax
from jax.experimental import pallas as pl
from jax.experimental.pallas import tpu as pltpu
from jax.experimental.pallas import tpu_sc as plsc

N = 10000
D = 128
E = 320000
NC = 2
NS = 16
NW = NC * NS
K = 128
EPW = E // NW
NH = 3
CH = 26
TAIL = EPW - NH * CH * K
ROWS_PT = 640
NPAD = ROWS_PT * NS
LAST = N - (NS - 1) * ROWS_PT

_MESH = plsc.VectorSubcoreMesh(core_axis_name="c", subcore_axis_name="s")


def _sc_agg_body(with_deg, h_hbm, src_hbm, dst_hbm, src16_hbm, dst16_hbm,
                 agg_hbm, *rest):
    if with_deg:
        (deg_hbm, src_v, dst_v, src16_v, dst16_v, rows_v, rows16_v,
         ones_v, zdeg_v, agg_s, deg_s, g0, g1, s0, s1) = rest
    else:
        (src_v, dst_v, src16_v, dst16_v, rows_v, rows16_v,
         agg_s, g0, g1, s0, s1) = rest

    c = lax.axis_index("c")
    s = lax.axis_index("s")
    wid = c * NS + s
    base = s * ROWS_PT

    def _fill_row(r, _):
        for l in range(D // 16):
            rows_v[0, r, pl.ds(l * 16, 16)] = jnp.zeros((16,), jnp.float32)
        return 0
    lax.fori_loop(0, K, _fill_row, 0)
    for b in range(ROWS_PT // K):
        pltpu.sync_copy(rows_v.at[0], agg_s.at[pl.ds(base + b * K, K)])
    if with_deg:
        def _fill_ones(i, _):
            ones_v[pl.ds(i * 16, 16)] = jnp.full((16,), 1.0, jnp.float32)
            return 0
        lax.fori_loop(0, K // 16, _fill_ones, 0)

        def _fill_zdeg(i, _):
            zdeg_v[pl.ds(i * 16, 16)] = jnp.zeros((16,), jnp.float32)
            return 0
        lax.fori_loop(0, ROWS_PT // 16, _fill_zdeg, 0)
        pltpu.sync_copy(zdeg_v, deg_s.at[pl.ds(base, ROWS_PT)])
    plsc.subcore_barrier()

    gsems = (g0, g1)
    ssems = (s0, s1)
    for seg in range(NH):
        pltpu.sync_copy(src_hbm.at[wid, seg], src_v)
        pltpu.sync_copy(dst_hbm.at[wid, seg], dst_v)
        pltpu.async_copy(h_hbm.at[src_v.at[0]], rows_v.at[0], g0)
        pltpu.async_copy(h_hbm.at[src_v.at[1]], rows_v.at[1], g1)

        def _cycle(g, _):
            for b in range(2):
                j = g * 2 + b
                pltpu.make_async_copy(h_hbm.at[pl.ds(0, K)], rows_v.at[b],
                                      gsems[b]).wait()
                sd = pltpu.async_copy(rows_v.at[b], agg_s.at[dst_v.at[j]],
                                      ssems[b], add=True)
                if with_deg:
                    pltpu.sync_copy(ones_v, deg_s.at[dst_v.at[j]], add=True)
                sd.wait()

                @pl.when(j + 2 < CH)
                def _():
                    pltpu.async_copy(h_hbm.at[src_v.at[j + 2]], rows_v.at[b],
                                     gsems[b])
            return 0
        lax.fori_loop(0, CH // 2, _cycle, 0)

    pltpu.sync_copy(src16_hbm.at[wid], src16_v)
    pltpu.sync_copy(dst16_hbm.at[wid], dst16_v)
    pltpu.async_copy(h_hbm.at[src16_v], rows16_v, g1).wait()
    pltpu.sync_copy(rows16_v, agg_s.at[dst16_v], add=True)
    if with_deg:
        pltpu.sync_copy(ones_v.at[pl.ds(0, TAIL)], deg_s.at[dst16_v], add=True)
    plsc.subcore_barrier()

    if with_deg:
        pltpu.sync_copy(deg_s.at[pl.ds(base, ROWS_PT)], zdeg_v)

    @pl.when(s < NS - 1)
    def _():
        pltpu.sync_copy(agg_s.at[pl.ds(base, ROWS_PT)],
                        agg_hbm.at[c, pl.ds(base, ROWS_PT)])
        if with_deg:
            pltpu.sync_copy(zdeg_v, deg_hbm.at[pl.ds(c * N + base, ROWS_PT)])

    @pl.when(s == NS - 1)
    def _():
        pltpu.sync_copy(agg_s.at[pl.ds(base, LAST)],
                        agg_hbm.at[c, pl.ds(base, LAST)])
        if with_deg:
            pltpu.sync_copy(zdeg_v.at[pl.ds(0, LAST)],
                            deg_hbm.at[pl.ds(c * N + base, LAST)])


_IDX_SCRATCH = [
    pltpu.VMEM((CH, K), jnp.int32),
    pltpu.VMEM((CH, K), jnp.int32),
    pltpu.VMEM((TAIL,), jnp.int32),
    pltpu.VMEM((TAIL,), jnp.int32),
    pltpu.VMEM((2, K, D), jnp.float32),
    pltpu.VMEM((TAIL, D), jnp.float32),
]

_sc_agg_deg = functools.partial(
    pl.kernel,
    out_type=(jax.ShapeDtypeStruct((NC, N, D), jnp.float32),
              jax.ShapeDtypeStruct((NC * N,), jnp.float32)),
    mesh=_MESH,
    scratch_types=_IDX_SCRATCH + [
        pltpu.VMEM((K,), jnp.float32),
        pltpu.VMEM((ROWS_PT,), jnp.float32),
        pltpu.VMEM_SHARED((NPAD, D), jnp.float32),
        pltpu.VMEM_SHARED((NPAD,), jnp.float32),
    ] + [pltpu.SemaphoreType.DMA] * 4,
)(functools.partial(_sc_agg_body, True))

_sc_agg_nodeg = functools.partial(
    pl.kernel,
    out_type=jax.ShapeDtypeStruct((NC, N, D), jnp.float32),
    mesh=_MESH,
    scratch_types=_IDX_SCRATCH + [
        pltpu.VMEM_SHARED((NPAD, D), jnp.float32),
    ] + [pltpu.SemaphoreType.DMA] * 4,
)(functools.partial(_sc_agg_body, False))


R = 2000


def _tc_layer_body(relu, agg_ref, degt_ref, h_ref, wl_ref, bl_ref, wr_ref, o_ref):
    aggsum = agg_ref[0] + agg_ref[1]
    deg = degt_ref[:, 0] + degt_ref[:, 1]
    invd = 1.0 / jnp.maximum(deg, 1.0)
    m = aggsum * invd[:, None]
    out = lax.dot_general(m, wl_ref[...], (((1,), (1,)), ((), ())),
                          preferred_element_type=jnp.float32)
    out = out + bl_ref[...]
    out = out + lax.dot_general(h_ref[...], wr_ref[...], (((1,), (1,)), ((), ())),
                                preferred_element_type=jnp.float32)
    if relu:
        out = jnp.maximum(out, 0.0)
    o_ref[...] = out


def _tc_layer(relu):
    return pl.pallas_call(
        functools.partial(_tc_layer_body, relu),
        grid=(N // R,),
        in_specs=[
            pl.BlockSpec((NC, R, D), lambda i: (0, i, 0)),
            pl.BlockSpec((R, NC), lambda i: (i, 0)),
            pl.BlockSpec((R, D), lambda i: (i, 0)),
            pl.BlockSpec((D, D), lambda i: (0, 0)),
            pl.BlockSpec((1, D), lambda i: (0, 0)),
            pl.BlockSpec((D, D), lambda i: (0, 0)),
        ],
        out_specs=pl.BlockSpec((R, D), lambda i: (i, 0)),
        out_shape=jax.ShapeDtypeStruct((N, D), jnp.float32),
    )


def kernel(x, edge_index, Wl1, bl1, Wr1, Wl2, bl2, Wr2, Wl3, bl3, Wr3,
           Wl4, bl4, Wr4, Wl5, bl5, Wr5):
    src2 = edge_index[0].reshape(NW, EPW)
    dst2 = edge_index[1].reshape(NW, EPW)
    full = NH * CH * K
    srcp = src2[:, :full].reshape(NW, NH, CH, K)
    dstp = dst2[:, :full].reshape(NW, NH, CH, K)
    src16 = src2[:, full:]
    dst16 = dst2[:, full:]

    layers = [(Wl1, bl1, Wr1), (Wl2, bl2, Wr2), (Wl3, bl3, Wr3),
              (Wl4, bl4, Wr4), (Wl5, bl5, Wr5)]
    h = x
    degt = None
    for i, (Wl, bl, Wr) in enumerate(layers):
        if i == 0:
            agg, deg = _sc_agg_deg(h, srcp, dstp, src16, dst16)
            degt = deg.reshape(NC, N).T
        else:
            agg = _sc_agg_nodeg(h, srcp, dstp, src16, dst16)
        h = _tc_layer(i < 4)(agg, degt, h, Wl, bl.reshape(1, D), Wr)
    return h


# speedup vs baseline: 1.0555x; 1.0140x over previous
import functools

import jax
import jax.numpy as jnp
from jax import lax
from jax.experimental import pallas as pl
from jax.experimental.pallas import tpu as pltpu
from jax.experimental.pallas import tpu_sc as plsc

N = 10000
D = 128
E = 320000
NC = 2
NS = 16
NW = NC * NS
K = 128
EPW = E // NW
NH = 3
CH = 26
TAIL = EPW - NH * CH * K
ROWS_PT = 640
NPAD = ROWS_PT * NS
LAST = N - (NS - 1) * ROWS_PT

_MESH = plsc.VectorSubcoreMesh(core_axis_name="c", subcore_axis_name="s")


def _sc_agg_body(with_deg, h_hbm, src_hbm, dst_hbm, src16_hbm, dst16_hbm,
                 agg_hbm, *rest):
    if with_deg:
        (deg_hbm, src_v, dst_v, src16_v, dst16_v, rows_v, rows16_v,
         ones_v, zdeg_v, agg_s, deg_s, g0, g1, s0, s1, t0) = rest
    else:
        (src_v, dst_v, src16_v, dst16_v, rows_v, rows16_v,
         agg_s, g0, g1, s0, s1, t0) = rest
    gsems = (g0, g1)
    ssems = (s0, s1)

    c = lax.axis_index("c")
    s = lax.axis_index("s")
    wid = c * NS + s
    base = s * ROWS_PT

    pltpu.sync_copy(src_hbm.at[wid, 0], src_v)
    pltpu.sync_copy(dst_hbm.at[wid, 0], dst_v)
    pltpu.sync_copy(src16_hbm.at[wid], src16_v)
    pltpu.sync_copy(dst16_hbm.at[wid], dst16_v)
    pltpu.async_copy(h_hbm.at[src16_v], rows16_v, t0)
    pltpu.async_copy(h_hbm.at[src_v.at[1]], rows_v.at[1], g1)

    def _fill_row(r, _):
        for l in range(D // 16):
            rows_v[0, r, pl.ds(l * 16, 16)] = jnp.zeros((16,), jnp.float32)
        return 0
    lax.fori_loop(0, K, _fill_row, 0)
    for b in range(ROWS_PT // K):
        pltpu.async_copy(rows_v.at[0], agg_s.at[pl.ds(base + b * K, K)], s0)
    if with_deg:
        def _fill_ones(i, _):
            ones_v[pl.ds(i * 16, 16)] = jnp.full((16,), 1.0, jnp.float32)
            return 0
        lax.fori_loop(0, K // 16, _fill_ones, 0)

        def _fill_zdeg(i, _):
            zdeg_v[pl.ds(i * 16, 16)] = jnp.zeros((16,), jnp.float32)
            return 0
        lax.fori_loop(0, ROWS_PT // 16, _fill_zdeg, 0)
        pltpu.async_copy(zdeg_v, deg_s.at[pl.ds(base, ROWS_PT)], s1)
    for b in range(ROWS_PT // K):
        pltpu.make_async_copy(rows_v.at[0],
                              agg_s.at[pl.ds(base, K)], s0).wait()
    if with_deg:
        pltpu.make_async_copy(zdeg_v,
                              deg_s.at[pl.ds(base, ROWS_PT)], s1).wait()
    pltpu.async_copy(h_hbm.at[src_v.at[0]], rows_v.at[0], g0)
    plsc.subcore_barrier()

    for seg in range(NH):
        if seg > 0:
            pltpu.sync_copy(src_hbm.at[wid, seg], src_v)
            pltpu.sync_copy(dst_hbm.at[wid, seg], dst_v)
            pltpu.async_copy(h_hbm.at[src_v.at[0]], rows_v.at[0], g0)
            pltpu.async_copy(h_hbm.at[src_v.at[1]], rows_v.at[1], g1)

        def _cycle(g, _):
            for b in range(2):
                j = g * 2 + b
                pltpu.make_async_copy(h_hbm.at[pl.ds(0, K)], rows_v.at[b],
                                      gsems[b]).wait()
                sd = pltpu.async_copy(rows_v.at[b], agg_s.at[dst_v.at[j]],
                                      ssems[b], add=True)
                if with_deg:
                    pltpu.sync_copy(ones_v, deg_s.at[dst_v.at[j]], add=True)
                sd.wait()

                @pl.when(j + 2 < CH)
                def _():
                    pltpu.async_copy(h_hbm.at[src_v.at[j + 2]], rows_v.at[b],
                                     gsems[b])
            return 0
        lax.fori_loop(0, CH // 2, _cycle, 0)

    pltpu.make_async_copy(h_hbm.at[pl.ds(0, TAIL)], rows16_v, t0).wait()
    pltpu.sync_copy(rows16_v, agg_s.at[dst16_v], add=True)
    if with_deg:
        pltpu.sync_copy(ones_v.at[pl.ds(0, TAIL)], deg_s.at[dst16_v], add=True)
    plsc.subcore_barrier()

    if with_deg:
        pltpu.sync_copy(deg_s.at[pl.ds(base, ROWS_PT)], zdeg_v)

    @pl.when(s < NS - 1)
    def _():
        pltpu.sync_copy(agg_s.at[pl.ds(base, ROWS_PT)],
                        agg_hbm.at[c, pl.ds(base, ROWS_PT)])
        if with_deg:
            pltpu.sync_copy(zdeg_v, deg_hbm.at[pl.ds(c * N + base, ROWS_PT)])

    @pl.when(s == NS - 1)
    def _():
        pltpu.sync_copy(agg_s.at[pl.ds(base, LAST)],
                        agg_hbm.at[c, pl.ds(base, LAST)])
        if with_deg:
            pltpu.sync_copy(zdeg_v.at[pl.ds(0, LAST)],
                            deg_hbm.at[pl.ds(c * N + base, LAST)])


_IDX_SCRATCH = [
    pltpu.VMEM((CH, K), jnp.int32),
    pltpu.VMEM((CH, K), jnp.int32),
    pltpu.VMEM((TAIL,), jnp.int32),
    pltpu.VMEM((TAIL,), jnp.int32),
    pltpu.VMEM((2, K, D), jnp.float32),
    pltpu.VMEM((TAIL, D), jnp.float32),
]

_sc_agg_deg = functools.partial(
    pl.kernel,
    out_type=(jax.ShapeDtypeStruct((NC, N, D), jnp.float32),
              jax.ShapeDtypeStruct((NC * N,), jnp.float32)),
    mesh=_MESH,
    scratch_types=_IDX_SCRATCH + [
        pltpu.VMEM((K,), jnp.float32),
        pltpu.VMEM((ROWS_PT,), jnp.float32),
        pltpu.VMEM_SHARED((NPAD, D), jnp.float32),
        pltpu.VMEM_SHARED((NPAD,), jnp.float32),
    ] + [pltpu.SemaphoreType.DMA] * 5,
)(functools.partial(_sc_agg_body, True))

_sc_agg_nodeg = functools.partial(
    pl.kernel,
    out_type=jax.ShapeDtypeStruct((NC, N, D), jnp.float32),
    mesh=_MESH,
    scratch_types=_IDX_SCRATCH + [
        pltpu.VMEM_SHARED((NPAD, D), jnp.float32),
    ] + [pltpu.SemaphoreType.DMA] * 5,
)(functools.partial(_sc_agg_body, False))


R = 2000


def _tc_layer_body(relu, agg_ref, degt_ref, h_ref, wl_ref, bl_ref, wr_ref, o_ref):
    aggsum = agg_ref[0] + agg_ref[1]
    deg = degt_ref[:, 0] + degt_ref[:, 1]
    invd = 1.0 / jnp.maximum(deg, 1.0)
    m = aggsum * invd[:, None]
    out = lax.dot_general(m, wl_ref[...], (((1,), (1,)), ((), ())),
                          preferred_element_type=jnp.float32)
    out = out + bl_ref[...]
    out = out + lax.dot_general(h_ref[...], wr_ref[...], (((1,), (1,)), ((), ())),
                                preferred_element_type=jnp.float32)
    if relu:
        out = jnp.maximum(out, 0.0)
    o_ref[...] = out


def _tc_layer(relu):
    return pl.pallas_call(
        functools.partial(_tc_layer_body, relu),
        grid=(N // R,),
        in_specs=[
            pl.BlockSpec((NC, R, D), lambda i: (0, i, 0)),
            pl.BlockSpec((R, NC), lambda i: (i, 0)),
            pl.BlockSpec((R, D), lambda i: (i, 0)),
            pl.BlockSpec((D, D), lambda i: (0, 0)),
            pl.BlockSpec((1, D), lambda i: (0, 0)),
            pl.BlockSpec((D, D), lambda i: (0, 0)),
        ],
        out_specs=pl.BlockSpec((R, D), lambda i: (i, 0)),
        out_shape=jax.ShapeDtypeStruct((N, D), jnp.float32),
    )


def kernel(x, edge_index, Wl1, bl1, Wr1, Wl2, bl2, Wr2, Wl3, bl3, Wr3,
           Wl4, bl4, Wr4, Wl5, bl5, Wr5):
    src2 = edge_index[0].reshape(NW, EPW)
    dst2 = edge_index[1].reshape(NW, EPW)
    full = NH * CH * K
    srcp = src2[:, :full].reshape(NW, NH, CH, K)
    dstp = dst2[:, :full].reshape(NW, NH, CH, K)
    src16 = src2[:, full:]
    dst16 = dst2[:, full:]

    layers = [(Wl1, bl1, Wr1), (Wl2, bl2, Wr2), (Wl3, bl3, Wr3),
              (Wl4, bl4, Wr4), (Wl5, bl5, Wr5)]
    h = x
    degt = None
    for i, (Wl, bl, Wr) in enumerate(layers):
        if i == 0:
            agg, deg = _sc_agg_deg(h, srcp, dstp, src16, dst16)
            degt = deg.reshape(NC, N).T
        else:
            agg = _sc_agg_nodeg(h, srcp, dstp, src16, dst16)
        h = _tc_layer(i < 4)(agg, degt, h, Wl, bl.reshape(1, D), Wr)
    return h
